# baseline (speedup 1.0000x reference)
import functools

import jax
import jax.numpy as jnp
from jax import lax
from jax.experimental import pallas as pl
from jax.experimental.pallas import tpu as pltpu
from jax.experimental.pallas import tpu_sc as plsc

N = 10000
E = 320000
D = 128

NC = 2
NS = 16
NW = NC * NS
CHUNK = 64
NCHUNK = 160
EPW = NCHUNK * CHUNK
E_PAD = NW * EPW
SB = 16
NSB = NCHUNK // SB
NROWCHUNK = 157
CPT = 10
NTAIL = N - (NROWCHUNK - 1) * CHUNK


def _matmul_body(x_ref, w_ref, o_ref):
    o_ref[...] = jnp.dot(x_ref[...], w_ref[...],
                         preferred_element_type=jnp.float32)


def _matmul(x, w):
    blk = 2000
    grid = N // blk
    return pl.pallas_call(
        _matmul_body,
        grid=(grid,),
        in_specs=[
            pl.BlockSpec((blk, D), lambda i: (i, 0)),
            pl.BlockSpec((D, D), lambda i: (0, 0)),
        ],
        out_specs=pl.BlockSpec((blk, D), lambda i: (i, 0)),
        out_shape=jax.ShapeDtypeStruct((N, D), jnp.float32),
    )(x, w)


def _edge_body(support_hbm, src_hbm, dst_hbm, ew_hbm, out_hbm,
               src0, src1, dst0, dst1, ewr0, ewr1, buf0, buf1, acc,
               sem0, sem1, ssem0, ssem1):
    cid = lax.axis_index("c")
    sid = lax.axis_index("s")
    wid = sid * NC + cid

    def zrow(i, _):
        for q in range(D // 16):
            buf0[i, pl.ds(q * 16, 16)] = jnp.zeros((16,), jnp.float32)
        return 0
    lax.fori_loop(0, CHUNK, zrow, 0)

    for k in range(CPT):
        c = sid * CPT + k

        @pl.when(c < NROWCHUNK - 1)
        def _():
            pltpu.sync_copy(buf0, acc.at[pl.ds(c * CHUNK, CHUNK)])

        @pl.when(c == NROWCHUNK - 1)
        def _():
            pltpu.sync_copy(buf0.at[pl.ds(0, NTAIL)],
                            acc.at[pl.ds((NROWCHUNK - 1) * CHUNK, NTAIL)])

    plsc.subcore_barrier()

    def stage_copies(s, srcr, dstr, ewr, ssem):
        yield pltpu.async_copy(src_hbm.at[wid, pl.ds(s * SB, SB)], srcr, ssem)
        yield pltpu.async_copy(dst_hbm.at[wid, pl.ds(s * SB, SB)], dstr, ssem)
        yield pltpu.async_copy(
            ew_hbm.at[wid, pl.ds(s * SB * CHUNK, SB * CHUNK)], ewr, ssem)

    def start_stage(s, srcr, dstr, ewr, ssem):
        for _ in stage_copies(s, srcr, dstr, ewr, ssem):
            pass

    def wait_stage(s, srcr, dstr, ewr, ssem):
        pltpu.make_async_copy(
            src_hbm.at[wid, pl.ds(s * SB, SB)], srcr, ssem).wait()
        pltpu.make_async_copy(
            dst_hbm.at[wid, pl.ds(s * SB, SB)], dstr, ssem).wait()
        pltpu.make_async_copy(
            ew_hbm.at[wid, pl.ds(s * SB * CHUNK, SB * CHUNK)], ewr,
            ssem).wait()

    def scale(buf, ewr, j):
        def _srow(i, _):
            idx = jnp.full((16,), j * CHUNK + i, jnp.int32)
            w = plsc.load_gather(ewr, [idx])
            for q in range(D // 16):
                sl = pl.ds(q * 16, 16)
                buf[i, sl] = buf[i, sl] * w
            return 0
        lax.fori_loop(0, CHUNK, _srow, 0)

    def start_gather(idx_ref, buf, sem):
        return pltpu.async_copy(support_hbm.at[idx_ref], buf, sem)

    def wait_gather(idx_ref, buf, sem):
        pltpu.make_async_copy(support_hbm.at[idx_ref], buf, sem).wait()

    slots = ((src0, dst0, ewr0, ssem0), (src1, dst1, ewr1, ssem1))
    start_stage(0, *slots[0])
    start_stage(1, *slots[1])

    for s in range(NSB):
        srcr, dstr, ewr, ssem = slots[s % 2]
        wait_stage(s, srcr, dstr, ewr, ssem)

        start_gather(srcr.at[0], buf0, sem0)
        start_gather(srcr.at[1], buf1, sem1)

        def pipe(k, _):
            a = 2 * k
            b = 2 * k + 1

            wait_gather(srcr.at[a], buf0, sem0)
            scale(buf0, ewr, a)
            pltpu.sync_copy(buf0, acc.at[dstr.at[a]], add=True)

            @pl.when(k < SB // 2 - 1)
            def _():
                start_gather(srcr.at[a + 2], buf0, sem0)

            wait_gather(srcr.at[b], buf1, sem1)
            scale(buf1, ewr, b)
            pltpu.sync_copy(buf1, acc.at[dstr.at[b]], add=True)

            @pl.when(k < SB // 2 - 1)
            def _():
                start_gather(srcr.at[b + 2], buf1, sem1)

            return 0

        lax.fori_loop(0, SB // 2, pipe, 0)

        if s + 2 < NSB:
            start_stage(s + 2, srcr, dstr, ewr, ssem)

    plsc.subcore_barrier()

    for k in range(CPT):
        c = sid * CPT + k

        @pl.when(c < NROWCHUNK - 1)
        def _():
            pltpu.sync_copy(acc.at[pl.ds(c * CHUNK, CHUNK)],
                            out_hbm.at[cid, pl.ds(c * CHUNK, CHUNK)])

        @pl.when(c == NROWCHUNK - 1)
        def _():
            pltpu.sync_copy(acc.at[pl.ds((NROWCHUNK - 1) * CHUNK, NTAIL)],
                            out_hbm.at[cid, pl.ds((NROWCHUNK - 1) * CHUNK,
                                                  NTAIL)])


def _edge_agg(support, src_p, dst_p, ew_p):
    mesh = plsc.VectorSubcoreMesh(core_axis_name="c", subcore_axis_name="s")
    k = functools.partial(
        pl.kernel,
        mesh=mesh,
        out_type=jax.ShapeDtypeStruct((NC, N, D), jnp.float32),
        scratch_types=[
            pltpu.VMEM((SB, CHUNK), jnp.int32),
            pltpu.VMEM((SB, CHUNK), jnp.int32),
            pltpu.VMEM((SB, CHUNK), jnp.int32),
            pltpu.VMEM((SB, CHUNK), jnp.int32),
            pltpu.VMEM((SB * CHUNK,), jnp.float32),
            pltpu.VMEM((SB * CHUNK,), jnp.float32),
            pltpu.VMEM((CHUNK, D), jnp.float32),
            pltpu.VMEM((CHUNK, D), jnp.float32),
            pltpu.VMEM_SHARED((N, D), jnp.float32),
            pltpu.SemaphoreType.DMA,
            pltpu.SemaphoreType.DMA,
            pltpu.SemaphoreType.DMA,
            pltpu.SemaphoreType.DMA,
        ],
        compiler_params=pltpu.CompilerParams(needs_layout_passes=False),
    )(_edge_body)
    return k(support, src_p, dst_p, ew_p)


def _combine_body(s_ref, p_ref, t_ref, o_ref):
    t = t_ref[0, 0]
    o_ref[...] = (1.0 - t) * s_ref[...] + t * (p_ref[0] + p_ref[1])


def _combine(support, partials, t):
    blk = 2000
    grid = N // blk
    t2 = t.reshape(1, 1)
    return pl.pallas_call(
        _combine_body,
        grid=(grid,),
        in_specs=[
            pl.BlockSpec((blk, D), lambda i: (i, 0)),
            pl.BlockSpec((NC, blk, D), lambda i: (0, i, 0)),
            pl.BlockSpec(memory_space=pltpu.MemorySpace.SMEM),
        ],
        out_specs=pl.BlockSpec((blk, D), lambda i: (i, 0)),
        out_shape=jax.ShapeDtypeStruct((N, D), jnp.float32),
    )(support, partials, t2)


def kernel(x, edge_index, edge_weight, weight, t):
    support = _matmul(x, weight)

    pad = E_PAD - E
    src = jnp.concatenate(
        [edge_index[0], jnp.zeros((pad,), jnp.int32)]).reshape(NW, NCHUNK, CHUNK)
    dst = jnp.concatenate(
        [edge_index[1], jnp.zeros((pad,), jnp.int32)]).reshape(NW, NCHUNK, CHUNK)
    ew = jnp.concatenate(
        [edge_weight, jnp.zeros((pad,), jnp.float32)]).reshape(NW, EPW)

    partials = _edge_agg(support, src, dst, ew)
    return _combine(support, partials, t)


# speedup vs baseline: 1.4289x; 1.4289x over previous
import functools

import jax
import jax.numpy as jnp
from jax import lax
from jax.experimental import pallas as pl
from jax.experimental.pallas import tpu as pltpu
from jax.experimental.pallas import tpu_sc as plsc

N = 10000
E = 320000
D = 128

NC = 2
NS = 16
NW = NC * NS
CHUNK = 128
NCHUNK = 79
EPW = NCHUNK * CHUNK
E_PAD = NW * EPW
NROWCHUNK = 79
CPT = 5


def _matmul_body(x_ref, w_ref, o_ref):
    o_ref[...] = jnp.dot(x_ref[...], w_ref[...],
                         preferred_element_type=jnp.float32)


def _matmul(x, w):
    blk = 2000
    grid = N // blk
    return pl.pallas_call(
        _matmul_body,
        grid=(grid,),
        in_specs=[
            pl.BlockSpec((blk, D), lambda i: (i, 0)),
            pl.BlockSpec((D, D), lambda i: (0, 0)),
        ],
        out_specs=pl.BlockSpec((blk, D), lambda i: (i, 0)),
        out_shape=jax.ShapeDtypeStruct((N, D), jnp.float32),
    )(x, w)


def _edge_body(support_hbm, src_hbm, dst_hbm, ew_hbm, out_hbm,
               src_v, dst_v, ew_v, buf, acc, gsem):
    cid = lax.axis_index("c")
    sid = lax.axis_index("s")
    wid = sid * NC + cid

    def zrow(i, _):
        for q in range(D // 16):
            buf[i, pl.ds(q * 16, 16)] = jnp.zeros((16,), jnp.float32)
        return 0
    lax.fori_loop(0, CHUNK, zrow, 0)

    for k in range(CPT):
        c = sid * CPT + k

        @pl.when(c < NROWCHUNK - 1)
        def _():
            pltpu.sync_copy(buf, acc.at[pl.ds(c * CHUNK, CHUNK)])

        @pl.when(c == NROWCHUNK - 1)
        def _():
            pltpu.sync_copy(buf.at[pl.ds(0, N - (NROWCHUNK - 1) * CHUNK)],
                            acc.at[pl.ds((NROWCHUNK - 1) * CHUNK,
                                         N - (NROWCHUNK - 1) * CHUNK)])

    pltpu.sync_copy(src_hbm.at[wid], src_v)
    pltpu.sync_copy(dst_hbm.at[wid], dst_v)
    pltpu.sync_copy(ew_hbm.at[wid], ew_v)

    plsc.subcore_barrier()

    def chunk_body(j, _):
        pltpu.async_copy(support_hbm.at[src_v.at[j]], buf, gsem).wait()

        def scale(g, _):
            wv = ew_v[pl.ds(j * CHUNK + g * 16, 16)]
            for r in range(16):
                w = wv[r]
                for q in range(D // 16):
                    sl = pl.ds(q * 16, 16)
                    buf[g * 16 + r, sl] = buf[g * 16 + r, sl] * w
            return 0
        lax.fori_loop(0, CHUNK // 16, scale, 0)

        pltpu.sync_copy(buf, acc.at[dst_v.at[j]], add=True)
        return 0

    lax.fori_loop(0, NCHUNK, chunk_body, 0)

    plsc.subcore_barrier()

    for k in range(CPT):
        c = sid * CPT + k

        @pl.when(c < NROWCHUNK - 1)
        def _():
            pltpu.sync_copy(acc.at[pl.ds(c * CHUNK, CHUNK)],
                            out_hbm.at[cid, pl.ds(c * CHUNK, CHUNK)])

        @pl.when(c == NROWCHUNK - 1)
        def _():
            rem = N - (NROWCHUNK - 1) * CHUNK
            pltpu.sync_copy(acc.at[pl.ds((NROWCHUNK - 1) * CHUNK, rem)],
                            out_hbm.at[cid, pl.ds((NROWCHUNK - 1) * CHUNK,
                                                  rem)])


def _edge_agg(support, src_p, dst_p, ew_p):
    mesh = plsc.VectorSubcoreMesh(core_axis_name="c", subcore_axis_name="s")
    k = functools.partial(
        pl.kernel,
        mesh=mesh,
        out_type=jax.ShapeDtypeStruct((NC, N, D), jnp.float32),
        scratch_types=[
            pltpu.VMEM((NCHUNK, CHUNK), jnp.int32),
            pltpu.VMEM((NCHUNK, CHUNK), jnp.int32),
            pltpu.VMEM((EPW,), jnp.float32),
            pltpu.VMEM((CHUNK, D), jnp.float32),
            pltpu.VMEM_SHARED((N, D), jnp.float32),
            pltpu.SemaphoreType.DMA,
        ],
        compiler_params=pltpu.CompilerParams(needs_layout_passes=False),
    )(_edge_body)
    return k(support, src_p, dst_p, ew_p)


def _combine_body(s_ref, p_ref, t_ref, o_ref):
    t = t_ref[0, 0]
    o_ref[...] = (1.0 - t) * s_ref[...] + t * (p_ref[0] + p_ref[1])


def _combine(support, partials, t):
    blk = 2000
    grid = N // blk
    t2 = t.reshape(1, 1)
    return pl.pallas_call(
        _combine_body,
        grid=(grid,),
        in_specs=[
            pl.BlockSpec((blk, D), lambda i: (i, 0)),
            pl.BlockSpec((NC, blk, D), lambda i: (0, i, 0)),
            pl.BlockSpec(memory_space=pltpu.MemorySpace.SMEM),
        ],
        out_specs=pl.BlockSpec((blk, D), lambda i: (i, 0)),
        out_shape=jax.ShapeDtypeStruct((N, D), jnp.float32),
    )(support, partials, t2)


def kernel(x, edge_index, edge_weight, weight, t):
    support = _matmul(x, weight)

    pad = E_PAD - E
    src = jnp.concatenate(
        [edge_index[0], jnp.zeros((pad,), jnp.int32)]).reshape(NW, NCHUNK, CHUNK)
    dst = jnp.concatenate(
        [edge_index[1], jnp.zeros((pad,), jnp.int32)]).reshape(NW, NCHUNK, CHUNK)
    ew = jnp.concatenate(
        [edge_weight, jnp.zeros((pad,), jnp.float32)]).reshape(NW, EPW)

    partials = _edge_agg(support, src, dst, ew)
    return _combine(support, partials, t)
